# baseline (speedup 1.0000x reference)
import functools

import jax
import jax.numpy as jnp
from jax import lax
from jax.experimental import pallas as pl
from jax.experimental.pallas import tpu as pltpu
from jax.experimental.pallas import tpu_sc as plsc

_NC = 2
_NS = 16
_NW = _NC * _NS
_L = 16


@functools.cache
def _build_sc_gather_dot(B, E):
    bpw = B // _NW
    n16 = bpw // _L
    ne = E // _L
    mesh = plsc.VectorSubcoreMesh(core_axis_name="c", subcore_axis_name="s")

    @functools.partial(
        pl.kernel,
        mesh=mesh,
        compiler_params=pltpu.CompilerParams(
            use_tc_tiling_on_sc=False,
            disable_bounds_checks=True,
            disable_semaphore_checks=True,
        ),
        out_type=(
            jax.ShapeDtypeStruct((_NW, _L), jnp.float32),
            jax.ShapeDtypeStruct((B,), jnp.float32),
        ),
        scratch_types=[
            pltpu.VMEM((bpw,), jnp.int32),
            pltpu.VMEM((bpw,), jnp.int32),
            pltpu.VMEM((bpw, E), jnp.float32),
            pltpu.VMEM((bpw, E), jnp.float32),
            pltpu.VMEM((bpw,), jnp.float32),
            pltpu.VMEM((bpw,), jnp.float32),
            pltpu.VMEM((bpw,), jnp.float32),
            pltpu.VMEM((_L,), jnp.float32),
            pltpu.SemaphoreType.DMA,
        ],
    )
    def sc_kernel(uid_hbm, mid_hbm, uemb_hbm, ubias_hbm, memb_hbm, mbias_hbm,
                  part_hbm, bsum_hbm,
                  uidx_v, midx_v, urows_v, mrows_v, ub_v, mb_v,
                  bsum_v, pacc_v, sem):
        wid = lax.axis_index("s") * _NC + lax.axis_index("c")
        base = wid * bpw
        pltpu.sync_copy(uid_hbm.at[pl.ds(base, bpw)], uidx_v)
        pltpu.sync_copy(mid_hbm.at[pl.ds(base, bpw)], midx_v)

        nck = bpw // 128
        row_copies = []
        bias_copies = []
        for k in range(nck):
            s128 = pl.ds(k * 128, 128)
            row_copies.append(pltpu.async_copy(
                uemb_hbm.at[uidx_v.at[s128]], urows_v.at[s128, :], sem))
            row_copies.append(pltpu.async_copy(
                memb_hbm.at[midx_v.at[s128]], mrows_v.at[s128, :], sem))
            bias_copies.append(pltpu.async_copy(
                ubias_hbm.at[uidx_v.at[s128]], ub_v.at[s128], sem))
            bias_copies.append(pltpu.async_copy(
                mbias_hbm.at[midx_v.at[s128]], mb_v.at[s128], sem))

        for c in bias_copies:
            c.wait()

        def bstep(i, c):
            s = pl.ds(i * _L, _L)
            bsum_v[s] = ub_v[s] + mb_v[s]
            return c

        lax.fori_loop(0, n16, bstep, 0)
        pltpu.sync_copy(bsum_v, bsum_hbm.at[pl.ds(base, bpw)])

        for c in row_copies:
            c.wait()

        zero = jnp.zeros((_L,), jnp.float32)

        def dotstep(j, accs):
            out = []
            for h in range(ne):
                u = urows_v[j, pl.ds(h * _L, _L)]
                m = mrows_v[j, pl.ds(h * _L, _L)]
                out.append(accs[h] + u * m)
            return tuple(out)

        accs = lax.fori_loop(0, bpw, dotstep, (zero,) * ne)
        total = accs[0]
        for h in range(1, ne):
            total = total + accs[h]
        pacc_v[...] = total
        pltpu.sync_copy(pacc_v, part_hbm.at[wid])

    return sc_kernel


def _finish_body(part_ref, bias_ref, out_ref):
    total = jnp.sum(part_ref[...])
    out_ref[...] = jax.nn.sigmoid(bias_ref[...] + total)


def kernel(inputs, user_embedding, user_bias, movie_embedding, movie_bias):
    B = inputs.shape[0]
    U, E = user_embedding.shape
    M = movie_embedding.shape[0]
    sc = _build_sc_gather_dot(B, E)
    part, bsum = sc(inputs[:, 0], inputs[:, 1], user_embedding,
                    user_bias.reshape(U), movie_embedding, movie_bias.reshape(M))
    rows = 128
    cols = B // rows
    out = pl.pallas_call(
        _finish_body,
        out_shape=jax.ShapeDtypeStruct((rows, cols), jnp.float32),
    )(part, bsum.reshape(rows, cols))
    return out.reshape(B, 1)
